# baseline (speedup 1.0000x reference)
import functools

import jax
import jax.numpy as jnp
from jax.experimental import pallas as pl
from jax.experimental.pallas import tpu as pltpu

NSEG = 16
FB = 1024


def _onehots(idx, nseg):
    cols = jax.lax.broadcasted_iota(jnp.int32, (idx.shape[0], nseg), 1)
    oh = (idx[:, None] == cols).astype(jnp.float32)
    rows = jax.lax.broadcasted_iota(jnp.int32, (nseg, idx.shape[0]), 0)
    oh_t = (rows == idx[None, :]).astype(jnp.float32)
    return oh, oh_t


def _pass1_body(idx_ref, w_ref, x_ref, gc_ref, cnt_ref):
    i = pl.program_id(0)
    nb = pl.num_programs(0)
    J = x_ref.shape[0]
    fb = x_ref.shape[1]

    idx = idx_ref[pl.ds(i * fb, fb)]
    oh, oh_t = _onehots(idx, NSEG)

    @pl.when(i == 0)
    def _init():
        gc_ref[...] = jnp.zeros_like(gc_ref)
        cnt_ref[...] = jnp.zeros_like(cnt_ref)

    cnt_ref[...] += jnp.sum(oh, axis=0)[None, :]
    for j in range(J):
        xj = x_ref[j]
        gc_ref[j] += jax.lax.dot_general(
            oh_t, xj, (((1,), (0,)), ((), ())),
            preferred_element_type=jnp.float32)

    @pl.when(i == nb - 1)
    def _finalize():
        inv = 1.0 / jnp.clip(cnt_ref[0, :], 1.0, None)
        w = w_ref[...]
        for j in range(J):
            mean_j = gc_ref[j] * inv[:, None]
            gc_ref[j] = jnp.tanh(
                jax.lax.dot_general(mean_j, w, (((1,), (0,)), ((), ())),
                                    preferred_element_type=jnp.float32))


def _pass2_body(idx_ref, gc_ref, cnt_ref, x_ref, out_ref):
    i = pl.program_id(0)
    nb = pl.num_programs(0)
    J = x_ref.shape[0]
    fb = x_ref.shape[1]

    idx = idx_ref[pl.ds(i * fb, fb)]
    oh, oh_t = _onehots(idx, NSEG)

    @pl.when(i == 0)
    def _init():
        out_ref[...] = jnp.zeros_like(out_ref)

    for j in range(J):
        xj = x_ref[j]
        scores = jax.lax.dot_general(
            xj, gc_ref[j], (((1,), (1,)), ((), ())),
            preferred_element_type=jnp.float32)
        gate = jax.nn.sigmoid(
            jnp.sum(scores * oh, axis=-1, keepdims=True))
        out_ref[j] += jax.lax.dot_general(
            oh_t, xj * gate, (((1,), (0,)), ((), ())),
            preferred_element_type=jnp.float32)

    @pl.when(i == nb - 1)
    def _finalize():
        inv = 1.0 / jnp.clip(cnt_ref[0, :], 1.0, None)
        out_ref[...] = out_ref[...] * inv[None, :, None]


@jax.jit
def kernel(x, batch_index, weight):
    J, F, C = x.shape
    idx = batch_index.astype(jnp.int32)
    nb = F // FB

    gc, cnt = pl.pallas_call(
        _pass1_body,
        grid=(nb,),
        in_specs=[
            pl.BlockSpec((F,), lambda i: (0,)),
            pl.BlockSpec((C, C), lambda i: (0, 0)),
            pl.BlockSpec((J, FB, C), lambda i: (0, i, 0)),
        ],
        out_specs=[
            pl.BlockSpec((J, NSEG, C), lambda i: (0, 0, 0)),
            pl.BlockSpec((1, NSEG), lambda i: (0, 0)),
        ],
        out_shape=[
            jax.ShapeDtypeStruct((J, NSEG, C), jnp.float32),
            jax.ShapeDtypeStruct((1, NSEG), jnp.float32),
        ],
    )(idx, weight, x)

    out = pl.pallas_call(
        _pass2_body,
        grid=(nb,),
        in_specs=[
            pl.BlockSpec((F,), lambda i: (0,)),
            pl.BlockSpec((J, NSEG, C), lambda i: (0, 0, 0)),
            pl.BlockSpec((1, NSEG), lambda i: (0, 0)),
            pl.BlockSpec((J, FB, C), lambda i: (0, i, 0)),
        ],
        out_specs=pl.BlockSpec((J, NSEG, C), lambda i: (0, 0, 0)),
        out_shape=jax.ShapeDtypeStruct((J, NSEG, C), jnp.float32),
    )(idx, gc, cnt, x)
    return out


# speedup vs baseline: 1.4223x; 1.4223x over previous
import functools

import jax
import jax.numpy as jnp
from jax.experimental import pallas as pl
from jax.experimental.pallas import tpu as pltpu

NSEG = 16
FB = 1024


def _onehots(idx, nseg):
    cols = jax.lax.broadcasted_iota(jnp.int32, (idx.shape[0], nseg), 1)
    oh = (idx[:, None] == cols).astype(jnp.float32)
    rows = jax.lax.broadcasted_iota(jnp.int32, (nseg, idx.shape[0]), 0)
    oh_t = (rows == idx[None, :]).astype(jnp.float32)
    return oh, oh_t


def _pass1_body(idx_ref, w_ref, x_ref, gc_ref, cnt_ref):
    i = pl.program_id(0)
    nb = pl.num_programs(0)
    J = x_ref.shape[0]
    fb = x_ref.shape[1]

    idx = idx_ref[pl.ds(i * fb, fb)]
    oh, oh_t = _onehots(idx, NSEG)

    @pl.when(i == 0)
    def _init():
        gc_ref[...] = jnp.zeros_like(gc_ref)
        cnt_ref[...] = jnp.zeros_like(cnt_ref)

    cnt_ref[...] += jnp.sum(oh, axis=0)[None, :]
    for j in range(J):
        xj = x_ref[j]
        gc_ref[j] += jax.lax.dot_general(
            oh_t, xj, (((1,), (0,)), ((), ())),
            preferred_element_type=jnp.float32)

    @pl.when(i == nb - 1)
    def _finalize():
        inv = 1.0 / jnp.clip(cnt_ref[0, :], 1.0, None)
        w = w_ref[...]
        for j in range(J):
            mean_j = gc_ref[j] * inv[:, None]
            gc_ref[j] = jnp.tanh(
                jax.lax.dot_general(mean_j, w, (((1,), (0,)), ((), ())),
                                    preferred_element_type=jnp.float32))


def _pass2_body(idx_ref, gc_ref, cnt_ref, x_ref, out_ref):
    i = pl.program_id(0)
    nb = pl.num_programs(0)
    J = x_ref.shape[0]
    fb = x_ref.shape[1]

    idx = idx_ref[pl.ds(i * fb, fb)]
    oh, oh_t = _onehots(idx, NSEG)

    @pl.when(i == 0)
    def _init():
        out_ref[...] = jnp.zeros_like(out_ref)

    del oh
    for j in range(J):
        xj = x_ref[j]
        scores = jax.lax.dot_general(
            gc_ref[j], xj, (((1,), (1,)), ((), ())),
            preferred_element_type=jnp.float32)
        gate = jax.nn.sigmoid(
            jnp.sum(scores * oh_t, axis=0, keepdims=True))
        out_ref[j] += jax.lax.dot_general(
            oh_t * gate, xj, (((1,), (0,)), ((), ())),
            preferred_element_type=jnp.float32)

    @pl.when(i == nb - 1)
    def _finalize():
        inv = 1.0 / jnp.clip(cnt_ref[0, :], 1.0, None)
        out_ref[...] = out_ref[...] * inv[None, :, None]


@jax.jit
def kernel(x, batch_index, weight):
    J, F, C = x.shape
    idx = batch_index.astype(jnp.int32)
    nb = F // FB

    gc, cnt = pl.pallas_call(
        _pass1_body,
        grid=(nb,),
        in_specs=[
            pl.BlockSpec((F,), lambda i: (0,)),
            pl.BlockSpec((C, C), lambda i: (0, 0)),
            pl.BlockSpec((J, FB, C), lambda i: (0, i, 0)),
        ],
        out_specs=[
            pl.BlockSpec((J, NSEG, C), lambda i: (0, 0, 0)),
            pl.BlockSpec((1, NSEG), lambda i: (0, 0)),
        ],
        out_shape=[
            jax.ShapeDtypeStruct((J, NSEG, C), jnp.float32),
            jax.ShapeDtypeStruct((1, NSEG), jnp.float32),
        ],
    )(idx, weight, x)

    out = pl.pallas_call(
        _pass2_body,
        grid=(nb,),
        in_specs=[
            pl.BlockSpec((F,), lambda i: (0,)),
            pl.BlockSpec((J, NSEG, C), lambda i: (0, 0, 0)),
            pl.BlockSpec((1, NSEG), lambda i: (0, 0)),
            pl.BlockSpec((J, FB, C), lambda i: (0, i, 0)),
        ],
        out_specs=pl.BlockSpec((J, NSEG, C), lambda i: (0, 0, 0)),
        out_shape=jax.ShapeDtypeStruct((J, NSEG, C), jnp.float32),
    )(idx, gc, cnt, x)
    return out


# speedup vs baseline: 1.6288x; 1.1451x over previous
import functools

import jax
import jax.numpy as jnp
from jax.experimental import pallas as pl
from jax.experimental.pallas import tpu as pltpu

NSEG = 16
FB = 2048


def _onehots(idx, nseg):
    cols = jax.lax.broadcasted_iota(jnp.int32, (idx.shape[0], nseg), 1)
    oh = (idx[:, None] == cols).astype(jnp.float32)
    rows = jax.lax.broadcasted_iota(jnp.int32, (nseg, idx.shape[0]), 0)
    oh_t = (rows == idx[None, :]).astype(jnp.float32)
    return oh, oh_t


def _pass1_body(idx_ref, w_ref, x_ref, gc_ref, cnt_ref):
    i = pl.program_id(0)
    nb = pl.num_programs(0)
    J = x_ref.shape[0]
    fb = x_ref.shape[1]

    idx = idx_ref[pl.ds(i * fb, fb)]
    oh, oh_t = _onehots(idx, NSEG)

    @pl.when(i == 0)
    def _init():
        gc_ref[...] = jnp.zeros_like(gc_ref)
        cnt_ref[...] = jnp.zeros_like(cnt_ref)

    cnt_ref[...] += jnp.sum(oh, axis=0)[None, :]
    for j in range(J):
        xj = x_ref[j]
        gc_ref[j] += jax.lax.dot_general(
            oh_t, xj, (((1,), (0,)), ((), ())),
            preferred_element_type=jnp.float32)

    @pl.when(i == nb - 1)
    def _finalize():
        inv = 1.0 / jnp.clip(cnt_ref[0, :], 1.0, None)
        w = w_ref[...]
        for j in range(J):
            mean_j = gc_ref[j] * inv[:, None]
            gc_ref[j] = jnp.tanh(
                jax.lax.dot_general(mean_j, w, (((1,), (0,)), ((), ())),
                                    preferred_element_type=jnp.float32))


def _pass2_body(idx_ref, gc_ref, cnt_ref, x_ref, out_ref):
    i = pl.program_id(0)
    nb = pl.num_programs(0)
    J = x_ref.shape[0]
    fb = x_ref.shape[1]

    idx = idx_ref[pl.ds(i * fb, fb)]
    oh, oh_t = _onehots(idx, NSEG)

    @pl.when(i == 0)
    def _init():
        out_ref[...] = jnp.zeros_like(out_ref)

    del oh
    for j in range(J):
        xj = x_ref[j]
        scores = jax.lax.dot_general(
            gc_ref[j], xj, (((1,), (1,)), ((), ())),
            preferred_element_type=jnp.float32)
        gate = jax.nn.sigmoid(
            jnp.sum(scores * oh_t, axis=0, keepdims=True))
        out_ref[j] += jax.lax.dot_general(
            oh_t * gate, xj, (((1,), (0,)), ((), ())),
            preferred_element_type=jnp.float32)

    @pl.when(i == nb - 1)
    def _finalize():
        inv = 1.0 / jnp.clip(cnt_ref[0, :], 1.0, None)
        out_ref[...] = out_ref[...] * inv[None, :, None]


@jax.jit
def kernel(x, batch_index, weight):
    J, F, C = x.shape
    idx = batch_index.astype(jnp.int32)
    nb = F // FB

    gc, cnt = pl.pallas_call(
        _pass1_body,
        grid=(nb,),
        in_specs=[
            pl.BlockSpec((F,), lambda i: (0,)),
            pl.BlockSpec((C, C), lambda i: (0, 0)),
            pl.BlockSpec((J, FB, C), lambda i: (0, i, 0)),
        ],
        out_specs=[
            pl.BlockSpec((J, NSEG, C), lambda i: (0, 0, 0)),
            pl.BlockSpec((1, NSEG), lambda i: (0, 0)),
        ],
        out_shape=[
            jax.ShapeDtypeStruct((J, NSEG, C), jnp.float32),
            jax.ShapeDtypeStruct((1, NSEG), jnp.float32),
        ],
    )(idx, weight, x)

    out = pl.pallas_call(
        _pass2_body,
        grid=(nb,),
        in_specs=[
            pl.BlockSpec((F,), lambda i: (0,)),
            pl.BlockSpec((J, NSEG, C), lambda i: (0, 0, 0)),
            pl.BlockSpec((1, NSEG), lambda i: (0, 0)),
            pl.BlockSpec((J, FB, C), lambda i: (0, i, 0)),
        ],
        out_specs=pl.BlockSpec((J, NSEG, C), lambda i: (0, 0, 0)),
        out_shape=jax.ShapeDtypeStruct((J, NSEG, C), jnp.float32),
    )(idx, gc, cnt, x)
    return out


# speedup vs baseline: 1.6557x; 1.0165x over previous
import jax
import jax.numpy as jnp
from jax.experimental import pallas as pl
from jax.experimental.pallas import tpu as pltpu

NSEG = 16
FB = 2048


def _onehot_t(idx, nseg):
    rows = jax.lax.broadcasted_iota(jnp.int32, (nseg, idx.shape[0]), 0)
    return (rows == idx[None, :]).astype(jnp.float32)


def _body(idx_ref, w_ref, x_ref, out_ref, gc_s, cnt_s):
    p = pl.program_id(0)
    i = pl.program_id(1)
    nb = pl.num_programs(1)
    J = x_ref.shape[0]
    fb = x_ref.shape[1]

    idx = idx_ref[pl.ds(i * fb, fb)]
    oh_t = _onehot_t(idx, NSEG)

    @pl.when(p == 0)
    def _pass1():
        @pl.when(i == 0)
        def _init():
            gc_s[...] = jnp.zeros_like(gc_s)
            cnt_s[...] = jnp.zeros_like(cnt_s)

        cnt_s[...] += jnp.sum(oh_t, axis=1)[None, :]
        for j in range(J):
            gc_s[j] += jax.lax.dot_general(
                oh_t, x_ref[j], (((1,), (0,)), ((), ())),
                preferred_element_type=jnp.float32)

        @pl.when(i == nb - 1)
        def _finalize_gc():
            inv = 1.0 / jnp.clip(cnt_s[0, :], 1.0, None)
            w = w_ref[...]
            for j in range(J):
                mean_j = gc_s[j] * inv[:, None]
                gc_s[j] = jnp.tanh(
                    jax.lax.dot_general(mean_j, w, (((1,), (0,)), ((), ())),
                                        preferred_element_type=jnp.float32))

    @pl.when(p == 1)
    def _pass2():
        @pl.when(i == 0)
        def _init():
            out_ref[...] = jnp.zeros_like(out_ref)

        for j in range(J):
            xj = x_ref[j]
            scores = jax.lax.dot_general(
                gc_s[j], xj, (((1,), (1,)), ((), ())),
                preferred_element_type=jnp.float32)
            gate = jax.nn.sigmoid(
                jnp.sum(scores * oh_t, axis=0, keepdims=True))
            out_ref[j] += jax.lax.dot_general(
                oh_t * gate, xj, (((1,), (0,)), ((), ())),
                preferred_element_type=jnp.float32)

        @pl.when(i == nb - 1)
        def _finalize_out():
            inv = 1.0 / jnp.clip(cnt_s[0, :], 1.0, None)
            out_ref[...] = out_ref[...] * inv[None, :, None]


@jax.jit
def kernel(x, batch_index, weight):
    J, F, C = x.shape
    idx = batch_index.astype(jnp.int32)
    nb = F // FB

    out = pl.pallas_call(
        _body,
        grid=(2, nb),
        in_specs=[
            pl.BlockSpec((F,), lambda p, i: (0,)),
            pl.BlockSpec((C, C), lambda p, i: (0, 0)),
            pl.BlockSpec((J, FB, C), lambda p, i: (0, i, 0)),
        ],
        out_specs=pl.BlockSpec((J, NSEG, C), lambda p, i: (0, 0, 0)),
        out_shape=jax.ShapeDtypeStruct((J, NSEG, C), jnp.float32),
        scratch_shapes=[
            pltpu.VMEM((J, NSEG, C), jnp.float32),
            pltpu.VMEM((1, NSEG), jnp.float32),
        ],
    )(idx, weight, x)
    return out
